# baseline (speedup 1.0000x reference)
import functools

import jax
import jax.numpy as jnp
from jax import lax
from jax.experimental import pallas as pl
from jax.experimental.pallas import tpu as pltpu
from jax.experimental.pallas import tpu_sc as plsc

N = 10000
E = 320000
D = 128

NC = 2
NS = 16
E_PER_SC = E // NC
E_PER_W = E_PER_SC // NS
N_PAD = 10240
RPT = N_PAD // NS

SEG_CHUNK = 128
SEG_ITERS = 80
SEG_HALF = SEG_ITERS // 2
E_PER_W_PAD = SEG_ITERS * SEG_CHUNK
ACC_ROWS = N + 8

HIST_CHUNK = 2000
HIST_ITERS = E_PER_W // HIST_CHUNK
HW = 16

_sc_mesh = plsc.VectorSubcoreMesh(core_axis_name="c", subcore_axis_name="s")
_sc_params = pltpu.CompilerParams(use_tc_tiling_on_sc=False)


@functools.partial(
    pl.kernel,
    out_type=jax.ShapeDtypeStruct((NC, N_PAD, HW), jnp.float32),
    mesh=_sc_mesh,
    scratch_types=[
        pltpu.VMEM((HIST_CHUNK,), jnp.int32),
        pltpu.VMEM((HIST_CHUNK, HW), jnp.float32),
        pltpu.VMEM((RPT, HW), jnp.float32),
        pltpu.VMEM_SHARED((N_PAD, HW), jnp.float32),
    ],
    compiler_params=_sc_params,
)
def _sc_hist(dst_hbm, out_hbm, dst_v, ones_v, zero_v, acc):
    c = lax.axis_index("c")
    s = lax.axis_index("s")

    one16 = jnp.ones((16,), jnp.float32)
    zer16 = jnp.zeros((16,), jnp.float32)

    @pl.loop(0, HIST_CHUNK)
    def _(i):
        ones_v[i, :] = one16

    @pl.loop(0, RPT)
    def _(i):
        zero_v[i, :] = zer16

    pltpu.sync_copy(zero_v, acc.at[pl.ds(s * RPT, RPT)])
    plsc.subcore_barrier()

    base = c * E_PER_SC + s * E_PER_W

    @pl.loop(0, HIST_ITERS)
    def _(i):
        pltpu.sync_copy(dst_hbm.at[pl.ds(base + i * HIST_CHUNK, HIST_CHUNK)],
                        dst_v)
        pltpu.sync_copy(ones_v, acc.at[dst_v], add=True)

    plsc.subcore_barrier()
    row0 = s * RPT
    pltpu.sync_copy(acc.at[pl.ds(row0, RPT)],
                    out_hbm.at[c, pl.ds(row0, RPT)])


@functools.partial(
    pl.kernel,
    out_type=jax.ShapeDtypeStruct((NC, ACC_ROWS, D), jnp.float32),
    mesh=_sc_mesh,
    scratch_types=(
        [pltpu.VMEM((SEG_HALF, SEG_CHUNK), jnp.int32),
         pltpu.VMEM((SEG_HALF, SEG_CHUNK), jnp.int32)]
        + [pltpu.VMEM((SEG_CHUNK, D), jnp.float32) for _ in range(2)]
        + [pltpu.VMEM_SHARED((ACC_ROWS, D), jnp.float32)]
        + [pltpu.SemaphoreType.DMA for _ in range(4)]
    ),
    compiler_params=_sc_params,
)
def _sc_seg(hs_hbm, src4_hbm, dst4_hbm, out_hbm,
            src_half, dst_half, r0, r1, acc, g0, g1, s0, s1):
    c = lax.axis_index("c")
    s = lax.axis_index("s")
    w = c * NS + s

    rows = (r0, r1)
    gsem = (g0, g1)
    ssem = (s0, s1)

    def start_gather(j, b):
        pltpu.async_copy(hs_hbm.at[src_half.at[j]], rows[b], gsem[b])

    def wait_gather(b):
        pltpu.make_async_copy(hs_hbm.at[src_half.at[0]], rows[b],
                              gsem[b]).wait()

    def start_scatter(j, b):
        pltpu.async_copy(rows[b], acc.at[dst_half.at[j]], ssem[b], add=True)

    def wait_scatter(b):
        pltpu.make_async_copy(rows[b], acc.at[dst_half.at[0]],
                              ssem[b]).wait()

    pltpu.sync_copy(src4_hbm.at[w, 0], src_half)
    pltpu.sync_copy(dst4_hbm.at[w, 0], dst_half)

    zer16 = jnp.zeros((16,), jnp.float32)

    @pl.loop(0, SEG_CHUNK)
    def _(i):
        for j in range(D // 16):
            r0[i, pl.ds(j * 16, 16)] = zer16

    zbase = s * (N // NS)
    nz = 625 // SEG_CHUNK
    for k in range(nz):
        pltpu.sync_copy(r0.at[pl.ds(0, SEG_CHUNK)],
                        acc.at[pl.ds(zbase + k * SEG_CHUNK, SEG_CHUNK)])
    pltpu.sync_copy(r0.at[pl.ds(0, 625 - nz * SEG_CHUNK)],
                    acc.at[pl.ds(zbase + nz * SEG_CHUNK, 625 - nz * SEG_CHUNK)])
    plsc.subcore_barrier()

    for h in range(2):
        if h == 1:
            pltpu.sync_copy(src4_hbm.at[w, 1], src_half)
            pltpu.sync_copy(dst4_hbm.at[w, 1], dst_half)
        start_gather(0, 0)
        start_gather(1, 1)
        wait_gather(0)
        start_scatter(0, 0)
        wait_gather(1)
        start_scatter(1, 1)

        @pl.loop(1, SEG_HALF // 2)
        def _(t):
            for b in range(2):
                j = 2 * t + b
                wait_scatter(b)
                start_gather(j, b)
            for b in range(2):
                wait_gather(b)
                start_scatter(2 * t + b, b)

        wait_scatter(0)
        wait_scatter(1)

    plsc.subcore_barrier()
    row0 = s * (N // NS)
    pltpu.sync_copy(acc.at[pl.ds(row0, N // NS)],
                    out_hbm.at[c, pl.ds(row0, N // NS)])


_BR = 1000
_GRID = N // _BR


def _prep_body(x_ref, w1_ref, d0_ref, d1_ref, hs_ref, dinvb_ref):
    deg = d0_ref[:, 0:1] + d1_ref[:, 0:1] + 1.0
    dinv = lax.rsqrt(deg)
    dinvb = jnp.broadcast_to(dinv, (_BR, D))
    h1 = jnp.dot(x_ref[...], w1_ref[...], preferred_element_type=jnp.float32)
    hs_ref[...] = h1 * dinvb
    dinvb_ref[...] = dinvb


def _mid_body(sa_ref, sb_ref, hs_ref, dinvb_ref, w2_ref, b1_ref,
              hs2_ref):
    dinvb = dinvb_ref[...]
    h = dinvb * (sa_ref[...] + sb_ref[...] + hs_ref[...]) + b1_ref[...]
    h = jnp.maximum(h, 0.0)
    h2 = jnp.dot(h, w2_ref[...], preferred_element_type=jnp.float32)
    hs2_ref[...] = h2 * dinvb


def _final_body(sa_ref, sb_ref, hs2_ref, dinvb_ref, b2_ref, out_ref):
    o = dinvb_ref[...] * (sa_ref[...] + sb_ref[...] + hs2_ref[...]) + b2_ref[...]
    m = jnp.max(o, axis=1, keepdims=True)
    z = o - m
    lse = jnp.log(jnp.sum(jnp.exp(z), axis=1, keepdims=True))
    out_ref[...] = z - lse


def _row_spec(w):
    return pl.BlockSpec((_BR, w), lambda i: (i, 0))


def _full_spec(h, w):
    return pl.BlockSpec((h, w), lambda i: (0, 0))


_prep = pl.pallas_call(
    _prep_body,
    grid=(_GRID,),
    in_specs=[_row_spec(D), _full_spec(D, D), _row_spec(HW), _row_spec(HW)],
    out_specs=[_row_spec(D), _row_spec(D)],
    out_shape=[jax.ShapeDtypeStruct((N, D), jnp.float32),
               jax.ShapeDtypeStruct((N, D), jnp.float32)],
)

_mid = pl.pallas_call(
    _mid_body,
    grid=(_GRID,),
    in_specs=[_row_spec(D), _row_spec(D), _row_spec(D), _row_spec(D),
              _full_spec(D, D), _full_spec(1, D)],
    out_specs=_row_spec(D),
    out_shape=jax.ShapeDtypeStruct((N, D), jnp.float32),
)

_final = pl.pallas_call(
    _final_body,
    grid=(_GRID,),
    in_specs=[_row_spec(D), _row_spec(D), _row_spec(D), _row_spec(D),
              _full_spec(1, D)],
    out_specs=_row_spec(D),
    out_shape=jax.ShapeDtypeStruct((N, D), jnp.float32),
)


@jax.jit
def kernel(x, edge_index, W1, b1, W2, b2):
    src = edge_index[0]
    dst = edge_index[1]

    nw = NC * NS
    pad = E_PER_W_PAD - E_PER_W
    src4 = jnp.pad(src.reshape(nw, E_PER_W),
                   ((0, 0), (0, pad))).reshape(nw, 2, SEG_HALF, SEG_CHUNK)
    dst4 = jnp.pad(dst.reshape(nw, E_PER_W), ((0, 0), (0, pad)),
                   constant_values=N).reshape(nw, 2, SEG_HALF, SEG_CHUNK)

    degp = _sc_hist(dst)
    hs1, dinvb = _prep(x, W1, degp[0], degp[1])

    seg1 = _sc_seg(hs1, src4, dst4)
    hs2 = _mid(seg1[0], seg1[1], hs1, dinvb, W2, b1.reshape(1, D))

    seg2 = _sc_seg(hs2, src4, dst4)
    return _final(seg2[0], seg2[1], hs2, dinvb, b2.reshape(1, D))


# speedup vs baseline: 1.6628x; 1.6628x over previous
import functools

import jax
import jax.numpy as jnp
from jax import lax
from jax.experimental import pallas as pl
from jax.experimental.pallas import tpu as pltpu
from jax.experimental.pallas import tpu_sc as plsc

N = 10000
E = 320000
D = 128

NC = 2
NS = 16
E_PER_SC = E // NC
E_PER_W = E_PER_SC // NS
N_PAD = 10240
RPT = N_PAD // NS

SEG_CHUNK = 120
SEG_ITERS = 84
E_PER_W_PAD = SEG_ITERS * SEG_CHUNK
ACC_ROWS = N + 8

HIST_CHUNK = 2000
HIST_ITERS = E_PER_W // HIST_CHUNK
HW = 16

_sc_mesh = plsc.VectorSubcoreMesh(core_axis_name="c", subcore_axis_name="s")
_sc_params = pltpu.CompilerParams(use_tc_tiling_on_sc=False)


@functools.partial(
    pl.kernel,
    out_type=jax.ShapeDtypeStruct((NC, N_PAD, HW), jnp.float32),
    mesh=_sc_mesh,
    scratch_types=[
        pltpu.VMEM((HIST_CHUNK,), jnp.int32),
        pltpu.VMEM((HIST_CHUNK, HW), jnp.float32),
        pltpu.VMEM((RPT, HW), jnp.float32),
        pltpu.VMEM_SHARED((N_PAD, HW), jnp.float32),
    ],
    compiler_params=_sc_params,
)
def _sc_hist(dst_hbm, out_hbm, dst_v, ones_v, zero_v, acc):
    c = lax.axis_index("c")
    s = lax.axis_index("s")

    one16 = jnp.ones((16,), jnp.float32)
    zer16 = jnp.zeros((16,), jnp.float32)

    @pl.loop(0, HIST_CHUNK)
    def _(i):
        ones_v[i, :] = one16

    @pl.loop(0, RPT)
    def _(i):
        zero_v[i, :] = zer16

    pltpu.sync_copy(zero_v, acc.at[pl.ds(s * RPT, RPT)])
    plsc.subcore_barrier()

    base = c * E_PER_SC + s * E_PER_W

    @pl.loop(0, HIST_ITERS)
    def _(i):
        pltpu.sync_copy(dst_hbm.at[pl.ds(base + i * HIST_CHUNK, HIST_CHUNK)],
                        dst_v)
        pltpu.sync_copy(ones_v, acc.at[dst_v], add=True)

    plsc.subcore_barrier()
    row0 = s * RPT
    pltpu.sync_copy(acc.at[pl.ds(row0, RPT)],
                    out_hbm.at[c, pl.ds(row0, RPT)])


@functools.partial(
    pl.kernel,
    out_type=jax.ShapeDtypeStruct((NC, ACC_ROWS, D), jnp.float32),
    mesh=_sc_mesh,
    scratch_types=(
        [pltpu.VMEM((SEG_ITERS, SEG_CHUNK), jnp.int32),
         pltpu.VMEM((SEG_ITERS, SEG_CHUNK), jnp.int32)]
        + [pltpu.VMEM((SEG_CHUNK, D), jnp.float32) for _ in range(2)]
        + [pltpu.VMEM_SHARED((ACC_ROWS, D), jnp.float32)]
        + [pltpu.SemaphoreType.DMA for _ in range(4)]
    ),
    compiler_params=_sc_params,
)
def _sc_seg(hs_hbm, src4_hbm, dst4_hbm, out_hbm,
            src_half, dst_half, r0, r1, acc, g0, g1, s0, s1):
    c = lax.axis_index("c")
    s = lax.axis_index("s")
    w = c * NS + s

    rows = (r0, r1)
    gsem = (g0, g1)
    ssem = (s0, s1)

    def start_gather(j, b):
        pltpu.async_copy(hs_hbm.at[src_half.at[j]], rows[b], gsem[b])

    def wait_gather(b):
        pltpu.make_async_copy(hs_hbm.at[src_half.at[0]], rows[b],
                              gsem[b]).wait()

    def start_scatter(j, b):
        pltpu.async_copy(rows[b], acc.at[dst_half.at[j]], ssem[b], add=True)

    def wait_scatter(b):
        pltpu.make_async_copy(rows[b], acc.at[dst_half.at[0]],
                              ssem[b]).wait()

    pltpu.sync_copy(src4_hbm.at[w], src_half)
    pltpu.sync_copy(dst4_hbm.at[w], dst_half)

    zer16 = jnp.zeros((16,), jnp.float32)

    @pl.loop(0, SEG_CHUNK)
    def _(i):
        for j in range(D // 16):
            r0[i, pl.ds(j * 16, 16)] = zer16

    zbase = s * (N // NS)
    nz = 625 // SEG_CHUNK
    for k in range(nz):
        pltpu.sync_copy(r0.at[pl.ds(0, SEG_CHUNK)],
                        acc.at[pl.ds(zbase + k * SEG_CHUNK, SEG_CHUNK)])
    pltpu.sync_copy(r0.at[pl.ds(0, 625 - nz * SEG_CHUNK)],
                    acc.at[pl.ds(zbase + nz * SEG_CHUNK, 625 - nz * SEG_CHUNK)])
    plsc.subcore_barrier()

    start_gather(0, 0)
    start_gather(1, 1)
    wait_gather(0)
    start_scatter(0, 0)
    wait_gather(1)
    start_scatter(1, 1)

    @pl.loop(1, SEG_ITERS // 2)
    def _(t):
        for b in range(2):
            j = 2 * t + b
            wait_scatter(b)
            start_gather(j, b)
        for b in range(2):
            wait_gather(b)
            start_scatter(2 * t + b, b)

    wait_scatter(0)
    wait_scatter(1)

    plsc.subcore_barrier()
    row0 = s * (N // NS)
    pltpu.sync_copy(acc.at[pl.ds(row0, N // NS)],
                    out_hbm.at[c, pl.ds(row0, N // NS)])


_BR = 1000
_GRID = N // _BR


def _prep_body(x_ref, w1_ref, d0_ref, d1_ref, hs_ref, dinvb_ref):
    deg = d0_ref[:, 0:1] + d1_ref[:, 0:1] + 1.0
    dinv = lax.rsqrt(deg)
    dinvb = jnp.broadcast_to(dinv, (_BR, D))
    h1 = jnp.dot(x_ref[...], w1_ref[...], preferred_element_type=jnp.float32)
    hs_ref[...] = h1 * dinvb
    dinvb_ref[...] = dinvb


def _mid_body(sa_ref, sb_ref, hs_ref, dinvb_ref, w2_ref, b1_ref,
              hs2_ref):
    dinvb = dinvb_ref[...]
    h = dinvb * (sa_ref[...] + sb_ref[...] + hs_ref[...]) + b1_ref[...]
    h = jnp.maximum(h, 0.0)
    h2 = jnp.dot(h, w2_ref[...], preferred_element_type=jnp.float32)
    hs2_ref[...] = h2 * dinvb


def _final_body(sa_ref, sb_ref, hs2_ref, dinvb_ref, b2_ref, out_ref):
    o = dinvb_ref[...] * (sa_ref[...] + sb_ref[...] + hs2_ref[...]) + b2_ref[...]
    m = jnp.max(o, axis=1, keepdims=True)
    z = o - m
    lse = jnp.log(jnp.sum(jnp.exp(z), axis=1, keepdims=True))
    out_ref[...] = z - lse


def _row_spec(w):
    return pl.BlockSpec((_BR, w), lambda i: (i, 0))


def _full_spec(h, w):
    return pl.BlockSpec((h, w), lambda i: (0, 0))


_prep = pl.pallas_call(
    _prep_body,
    grid=(_GRID,),
    in_specs=[_row_spec(D), _full_spec(D, D), _row_spec(HW), _row_spec(HW)],
    out_specs=[_row_spec(D), _row_spec(D)],
    out_shape=[jax.ShapeDtypeStruct((N, D), jnp.float32),
               jax.ShapeDtypeStruct((N, D), jnp.float32)],
)

_mid = pl.pallas_call(
    _mid_body,
    grid=(_GRID,),
    in_specs=[_row_spec(D), _row_spec(D), _row_spec(D), _row_spec(D),
              _full_spec(D, D), _full_spec(1, D)],
    out_specs=_row_spec(D),
    out_shape=jax.ShapeDtypeStruct((N, D), jnp.float32),
)

_final = pl.pallas_call(
    _final_body,
    grid=(_GRID,),
    in_specs=[_row_spec(D), _row_spec(D), _row_spec(D), _row_spec(D),
              _full_spec(1, D)],
    out_specs=_row_spec(D),
    out_shape=jax.ShapeDtypeStruct((N, D), jnp.float32),
)


@jax.jit
def kernel(x, edge_index, W1, b1, W2, b2):
    src = edge_index[0]
    dst = edge_index[1]

    nw = NC * NS
    pad = E_PER_W_PAD - E_PER_W
    src4 = jnp.pad(src.reshape(nw, E_PER_W),
                   ((0, 0), (0, pad))).reshape(nw, SEG_ITERS, SEG_CHUNK)
    dst4 = jnp.pad(dst.reshape(nw, E_PER_W), ((0, 0), (0, pad)),
                   constant_values=N).reshape(nw, SEG_ITERS, SEG_CHUNK)

    degp = _sc_hist(dst)
    hs1, dinvb = _prep(x, W1, degp[0], degp[1])

    seg1 = _sc_seg(hs1, src4, dst4)
    hs2 = _mid(seg1[0], seg1[1], hs1, dinvb, W2, b1.reshape(1, D))

    seg2 = _sc_seg(hs2, src4, dst4)
    return _final(seg2[0], seg2[1], hs2, dinvb, b2.reshape(1, D))


# speedup vs baseline: 2.3660x; 1.4229x over previous
import functools

import jax
import jax.numpy as jnp
from jax import lax
from jax.experimental import pallas as pl
from jax.experimental.pallas import tpu as pltpu
from jax.experimental.pallas import tpu_sc as plsc

N = 10000
E = 320000
D = 128

NC = 2
NS = 16
E_PER_SC = E // NC
E_PER_W = E_PER_SC // NS
N_PAD = 10240
RPT = N_PAD // NS

SEG_CHUNK = 100
SEG_ITERS = 100
E_PER_W_PAD = SEG_ITERS * SEG_CHUNK
ACC_ROWS = N + 8

HIST_CHUNK = 2000
HIST_ITERS = E_PER_W // HIST_CHUNK
HW = 16

_sc_mesh = plsc.VectorSubcoreMesh(core_axis_name="c", subcore_axis_name="s")
_sc_params = pltpu.CompilerParams(use_tc_tiling_on_sc=False)


@functools.partial(
    pl.kernel,
    out_type=jax.ShapeDtypeStruct((NC, N_PAD, HW), jnp.float32),
    mesh=_sc_mesh,
    scratch_types=[
        pltpu.VMEM((HIST_CHUNK,), jnp.int32),
        pltpu.VMEM((HIST_CHUNK, HW), jnp.float32),
        pltpu.VMEM((RPT, HW), jnp.float32),
        pltpu.VMEM_SHARED((N_PAD, HW), jnp.float32),
    ],
    compiler_params=_sc_params,
)
def _sc_hist(dst_hbm, out_hbm, dst_v, ones_v, zero_v, acc):
    c = lax.axis_index("c")
    s = lax.axis_index("s")

    one16 = jnp.ones((16,), jnp.float32)
    zer16 = jnp.zeros((16,), jnp.float32)

    @pl.loop(0, HIST_CHUNK)
    def _(i):
        ones_v[i, :] = one16

    @pl.loop(0, RPT)
    def _(i):
        zero_v[i, :] = zer16

    pltpu.sync_copy(zero_v, acc.at[pl.ds(s * RPT, RPT)])
    plsc.subcore_barrier()

    base = c * E_PER_SC + s * E_PER_W

    @pl.loop(0, HIST_ITERS)
    def _(i):
        pltpu.sync_copy(dst_hbm.at[pl.ds(base + i * HIST_CHUNK, HIST_CHUNK)],
                        dst_v)
        pltpu.sync_copy(ones_v, acc.at[dst_v], add=True)

    plsc.subcore_barrier()
    row0 = s * RPT
    pltpu.sync_copy(acc.at[pl.ds(row0, RPT)],
                    out_hbm.at[c, pl.ds(row0, RPT)])


@functools.partial(
    pl.kernel,
    out_type=jax.ShapeDtypeStruct((NC, ACC_ROWS, D), jnp.float32),
    mesh=_sc_mesh,
    scratch_types=(
        [pltpu.VMEM((SEG_ITERS, SEG_CHUNK), jnp.int32),
         pltpu.VMEM((SEG_ITERS, SEG_CHUNK), jnp.int32)]
        + [pltpu.VMEM((SEG_CHUNK, D), jnp.float32) for _ in range(2)]
        + [pltpu.VMEM_SHARED((ACC_ROWS, D), jnp.float32)]
        + [pltpu.SemaphoreType.DMA for _ in range(4)]
    ),
    compiler_params=_sc_params,
)
def _sc_seg(hs_hbm, src4_hbm, dst4_hbm, out_hbm,
            src_half, dst_half, r0, r1, acc, g0, g1, s0, s1):
    c = lax.axis_index("c")
    s = lax.axis_index("s")
    w = c * NS + s

    rows = (r0, r1)
    gsem = (g0, g1)
    ssem = (s0, s1)

    def start_gather(j, b):
        pltpu.async_copy(hs_hbm.at[src_half.at[j]], rows[b], gsem[b])

    def wait_gather(b):
        pltpu.make_async_copy(hs_hbm.at[src_half.at[0]], rows[b],
                              gsem[b]).wait()

    def start_scatter(j, b):
        pltpu.async_copy(rows[b], acc.at[dst_half.at[j]], ssem[b], add=True)

    def wait_scatter(b):
        pltpu.make_async_copy(rows[b], acc.at[dst_half.at[0]],
                              ssem[b]).wait()

    pltpu.sync_copy(src4_hbm.at[w], src_half)
    pltpu.sync_copy(dst4_hbm.at[w], dst_half)

    zer16 = jnp.zeros((16,), jnp.float32)

    @pl.loop(0, SEG_CHUNK)
    def _(i):
        for j in range(D // 16):
            r0[i, pl.ds(j * 16, 16)] = zer16

    zbase = s * (N // NS)
    nz = 625 // SEG_CHUNK
    for k in range(nz):
        pltpu.sync_copy(r0.at[pl.ds(0, SEG_CHUNK)],
                        acc.at[pl.ds(zbase + k * SEG_CHUNK, SEG_CHUNK)])
    pltpu.sync_copy(r0.at[pl.ds(0, 625 - nz * SEG_CHUNK)],
                    acc.at[pl.ds(zbase + nz * SEG_CHUNK, 625 - nz * SEG_CHUNK)])
    plsc.subcore_barrier()

    start_gather(0, 0)
    start_gather(1, 1)
    wait_gather(0)
    start_scatter(0, 0)
    wait_gather(1)
    start_scatter(1, 1)

    @pl.loop(1, SEG_ITERS // 2)
    def _(t):
        for b in range(2):
            j = 2 * t + b
            wait_scatter(b)
            start_gather(j, b)
        for b in range(2):
            wait_gather(b)
            start_scatter(2 * t + b, b)

    wait_scatter(0)
    wait_scatter(1)

    plsc.subcore_barrier()
    row0 = s * (N // NS)
    pltpu.sync_copy(acc.at[pl.ds(row0, N // NS)],
                    out_hbm.at[c, pl.ds(row0, N // NS)])


_BR = 1000
_GRID = N // _BR


def _prep_body(x_ref, w1_ref, d0_ref, d1_ref, hs_ref, dinvb_ref):
    deg = d0_ref[:, 0:1] + d1_ref[:, 0:1] + 1.0
    dinv = lax.rsqrt(deg)
    dinvb = jnp.broadcast_to(dinv, (_BR, D))
    h1 = jnp.dot(x_ref[...], w1_ref[...], preferred_element_type=jnp.float32)
    hs_ref[...] = h1 * dinvb
    dinvb_ref[...] = dinvb


def _mid_body(sa_ref, sb_ref, hs_ref, dinvb_ref, w2_ref, b1_ref,
              hs2_ref):
    dinvb = dinvb_ref[...]
    h = dinvb * (sa_ref[...] + sb_ref[...] + hs_ref[...]) + b1_ref[...]
    h = jnp.maximum(h, 0.0)
    h2 = jnp.dot(h, w2_ref[...], preferred_element_type=jnp.float32)
    hs2_ref[...] = h2 * dinvb


def _final_body(sa_ref, sb_ref, hs2_ref, dinvb_ref, b2_ref, out_ref):
    o = dinvb_ref[...] * (sa_ref[...] + sb_ref[...] + hs2_ref[...]) + b2_ref[...]
    m = jnp.max(o, axis=1, keepdims=True)
    z = o - m
    lse = jnp.log(jnp.sum(jnp.exp(z), axis=1, keepdims=True))
    out_ref[...] = z - lse


def _row_spec(w):
    return pl.BlockSpec((_BR, w), lambda i: (i, 0))


def _full_spec(h, w):
    return pl.BlockSpec((h, w), lambda i: (0, 0))


_prep = pl.pallas_call(
    _prep_body,
    grid=(_GRID,),
    in_specs=[_row_spec(D), _full_spec(D, D), _row_spec(HW), _row_spec(HW)],
    out_specs=[_row_spec(D), _row_spec(D)],
    out_shape=[jax.ShapeDtypeStruct((N, D), jnp.float32),
               jax.ShapeDtypeStruct((N, D), jnp.float32)],
)

_mid = pl.pallas_call(
    _mid_body,
    grid=(_GRID,),
    in_specs=[_row_spec(D), _row_spec(D), _row_spec(D), _row_spec(D),
              _full_spec(D, D), _full_spec(1, D)],
    out_specs=_row_spec(D),
    out_shape=jax.ShapeDtypeStruct((N, D), jnp.float32),
)

_final = pl.pallas_call(
    _final_body,
    grid=(_GRID,),
    in_specs=[_row_spec(D), _row_spec(D), _row_spec(D), _row_spec(D),
              _full_spec(1, D)],
    out_specs=_row_spec(D),
    out_shape=jax.ShapeDtypeStruct((N, D), jnp.float32),
)


@jax.jit
def kernel(x, edge_index, W1, b1, W2, b2):
    src = edge_index[0]
    dst = edge_index[1]

    nw = NC * NS
    pad = E_PER_W_PAD - E_PER_W
    src4 = jnp.pad(src.reshape(nw, E_PER_W),
                   ((0, 0), (0, pad))).reshape(nw, SEG_ITERS, SEG_CHUNK)
    dst4 = jnp.pad(dst.reshape(nw, E_PER_W), ((0, 0), (0, pad)),
                   constant_values=N).reshape(nw, SEG_ITERS, SEG_CHUNK)

    degp = _sc_hist(dst)
    hs1, dinvb = _prep(x, W1, degp[0], degp[1])

    seg1 = _sc_seg(hs1, src4, dst4)
    hs2 = _mid(seg1[0], seg1[1], hs1, dinvb, W2, b1.reshape(1, D))

    seg2 = _sc_seg(hs2, src4, dst4)
    return _final(seg2[0], seg2[1], hs2, dinvb, b2.reshape(1, D))
